# baseline (speedup 1.0000x reference)
import functools

import jax
import jax.numpy as jnp
from jax import lax
from jax.experimental import pallas as pl
from jax.experimental.pallas import tpu as pltpu
from jax.experimental.pallas import tpu_sc as plsc

M = 300000
B = 100000
NW = 32
L = 16

BW = 3136
SM = 9376
CH = 2000
GG = 112

_mesh = plsc.VectorSubcoreMesh(core_axis_name="c", subcore_axis_name="s")
_cparams = pltpu.CompilerParams(needs_layout_passes=False,
                                use_tc_tiling_on_sc=False)


def _wid():
    return lax.axis_index("c") * 16 + lax.axis_index("s")


@functools.partial(
    pl.kernel,
    out_type=jax.ShapeDtypeStruct((M,), jnp.int32),
    mesh=_mesh,
    compiler_params=_cparams,
    scratch_types=[
        pltpu.VMEM((B,), jnp.int32),
        pltpu.VMEM((SM,), jnp.int32),
    ],
)
def _k_scan(idxh, s_out, idx_v, s_v):
    wid = _wid()
    lanes = lax.iota(jnp.int32, L)
    sbase = pl.multiple_of(jnp.minimum(wid * SM, M - SM), 16)
    pltpu.sync_copy(idxh, idx_v)

    def chunk(c, carry):
        cb = pl.multiple_of(c * CH, 16)
        for v in range(CH // L):
            iv = idx_v[pl.ds(cb + v * L, L)]
            jv = lanes + (cb + v * L)
            sl = iv - sbase
            m = plsc.bitcast(sl, jnp.uint32) < jnp.uint32(SM)
            slc = jnp.where(m, sl, 0)
            plsc.store_scatter(s_v, [slc], jv, mask=m)
            g1 = plsc.load_gather(s_v, [slc], mask=m)
            m2 = m & (g1 < jv)
            plsc.store_scatter(s_v, [slc], jv, mask=m2)
        return carry

    lax.fori_loop(0, B // CH, chunk, 0)
    pltpu.sync_copy(s_v, s_out.at[pl.ds(sbase, SM)])


@functools.partial(
    pl.kernel,
    out_type=jax.ShapeDtypeStruct((B,), jnp.int32),
    mesh=_mesh,
    compiler_params=_cparams,
    scratch_types=[
        pltpu.VMEM((BW,), jnp.int32),
        pltpu.VMEM((BW,), jnp.int32),
        pltpu.SemaphoreType.DMA,
    ],
)
def _k_wsel(s_hbm, idxh, w_out, iv_v, wv_v, sem):
    wid = _wid()
    base = pl.multiple_of(jnp.minimum(wid * BW, B - BW), 16)
    pltpu.sync_copy(idxh.at[pl.ds(base, BW)], iv_v)
    ds = []
    for c in range(BW // GG):
        ds.append(pltpu.async_copy(
            s_hbm.at[iv_v.at[pl.ds(c * GG, GG)]],
            wv_v.at[pl.ds(c * GG, GG)], sem))
    for d in ds:
        d.wait()
    pltpu.sync_copy(wv_v, w_out.at[pl.ds(base, BW)])


@functools.partial(
    pl.kernel,
    out_type=jax.ShapeDtypeStruct((B * 10,), jnp.float32),
    mesh=_mesh,
    compiler_params=_cparams,
    scratch_types=[
        pltpu.VMEM((BW * 3,), jnp.float32),
        pltpu.VMEM((BW * 3,), jnp.float32),
        pltpu.VMEM((BW * 3,), jnp.float32),
        pltpu.VMEM((BW,), jnp.float32),
        pltpu.VMEM((BW * 10,), jnp.float32),
        pltpu.VMEM((L,), jnp.float32),
    ],
)
def _k_rows(allf, rows_out, pos_v, col_v, scl_v, opa_v, rows_v, t_v):
    wid = _wid()
    base = pl.multiple_of(jnp.minimum(wid * BW, B - BW), 16)
    lanes = lax.iota(jnp.int32, L)
    i3 = lanes * 3
    i10 = lanes * 10

    pltpu.sync_copy(allf.at[pl.ds(10 * B, L)], t_v)
    pltpu.sync_copy(allf.at[pl.ds(base * 3, BW * 3)], pos_v)
    pltpu.sync_copy(allf.at[pl.ds(3 * B + base * 3, BW * 3)], col_v)
    pltpu.sync_copy(allf.at[pl.ds(6 * B + base * 3, BW * 3)], scl_v)
    pltpu.sync_copy(allf.at[pl.ds(9 * B + base, BW)], opa_v)

    tv = t_v[...]
    zf = jnp.zeros((L,), jnp.float32)
    t = [jnp.broadcast_to(jnp.sum(jnp.where(lanes == k, tv, zf)), (L,))
         for k in range(12)]

    def grp(g, carry):
        r3 = g * (L * 3)
        r10 = g * (L * 10)
        src = i3 + r3
        px = plsc.load_gather(pos_v, [src])
        py = plsc.load_gather(pos_v, [src + 1])
        pz = plsc.load_gather(pos_v, [src + 2])
        pwx = t[0] * px + t[1] * py + t[2] * pz + t[3]
        pwy = t[4] * px + t[5] * py + t[6] * pz + t[7]
        pwz = t[8] * px + t[9] * py + t[10] * pz + t[11]
        ob = i10 + r10
        plsc.store_scatter(rows_v, [ob], pwx)
        plsc.store_scatter(rows_v, [ob + 1], pwy)
        plsc.store_scatter(rows_v, [ob + 2], pwz)
        for k in range(3):
            plsc.store_scatter(rows_v, [ob + 3 + k], plsc.load_gather(col_v, [src + k]))
        for k in range(3):
            plsc.store_scatter(rows_v, [ob + 6 + k], plsc.load_gather(scl_v, [src + k]))
        op = plsc.load_gather(opa_v, [lanes + g * L])
        plsc.store_scatter(rows_v, [ob + 9], op)
        return carry

    lax.fori_loop(0, BW // L, grp, 0)
    pltpu.sync_copy(rows_v, rows_out.at[pl.ds(base * 10, BW * 10)])


@functools.partial(
    pl.kernel,
    out_type=jax.ShapeDtypeStruct((B * 10,), jnp.float32),
    mesh=_mesh,
    compiler_params=_cparams,
    scratch_types=[
        pltpu.VMEM((BW,), jnp.int32),
        pltpu.VMEM((BW * 10,), jnp.int32),
        pltpu.VMEM((BW * 10,), jnp.float32),
        pltpu.SemaphoreType.DMA,
    ],
)
def _k_out(w_hbm, rowsf_hbm, out_hbm, wv_v, idx10_v, orow_v, sem):
    wid = _wid()
    base = pl.multiple_of(jnp.minimum(wid * BW, B - BW), 16)
    lanes = lax.iota(jnp.int32, L)
    i10 = lanes * 10

    pltpu.sync_copy(w_hbm.at[pl.ds(base, BW)], wv_v)

    def grp(g, carry):
        wv = plsc.load_gather(wv_v, [lanes + g * L])
        w10 = wv * 10
        ob = i10 + g * (L * 10)
        for c in range(10):
            plsc.store_scatter(idx10_v, [ob + c], w10 + c)
        return carry

    lax.fori_loop(0, BW // L, grp, 0)

    ds = []
    for c in range(BW * 10 // GG):
        ds.append(pltpu.async_copy(
            rowsf_hbm.at[idx10_v.at[pl.ds(c * GG, GG)]],
            orow_v.at[pl.ds(c * GG, GG)], sem))
    for d in ds:
        d.wait()

    pltpu.sync_copy(orow_v, out_hbm.at[pl.ds(base * 10, BW * 10)])


def kernel(mem_positions, mem_colors, mem_scales, mem_opacities, T_obj_world,
           positions, colors, scales, opacities, idx):
    idx32 = idx.astype(jnp.int32)
    s = _k_scan(idx32)
    w = _k_wsel(s, idx32)
    allf = jnp.concatenate([
        positions.reshape(-1), colors.reshape(-1), scales.reshape(-1),
        opacities.reshape(-1), T_obj_world.reshape(-1)])
    rows_flat = _k_rows(allf)
    return _k_out(w, rows_flat).reshape(B, 10)
